# baseline (speedup 1.0000x reference)
import functools

import jax
import jax.numpy as jnp
from jax.experimental import pallas as pl
from jax.experimental.pallas import tpu as pltpu

N = 10000
E = 40000
E2 = 120000
NUM_REL = 7
NUM_ANGLE = 8
NUM_GRAPHS = 32
EPS = 1e-5


def _round_up(x, m):
    return ((x + m - 1) // m) * m


def _mm_kernel(a_ref, b_ref, bias_ref, o_ref, *, relu):
    acc = jnp.dot(a_ref[...], b_ref[...], preferred_element_type=jnp.float32)
    acc = acc + bias_ref[...]
    if relu:
        acc = jnp.maximum(acc, 0.0)
    o_ref[...] = acc


def _matmul(a, b, bias, relu=False, bm=1024):
    m, k = a.shape
    k2, n = b.shape
    assert k == k2
    mp = _round_up(m, bm)
    kp = _round_up(k, 128)
    np_ = _round_up(n, 128)
    a = jnp.pad(a, ((0, mp - m), (0, kp - k)))
    b = jnp.pad(b, ((0, kp - k), (0, np_ - n)))
    bias = jnp.pad(bias, ((0, np_ - n),)).reshape(1, np_)
    out = pl.pallas_call(
        functools.partial(_mm_kernel, relu=relu),
        grid=(mp // bm,),
        in_specs=[
            pl.BlockSpec((bm, kp), lambda i: (i, 0)),
            pl.BlockSpec((kp, np_), lambda i: (0, 0)),
            pl.BlockSpec((1, np_), lambda i: (0, 0)),
        ],
        out_specs=pl.BlockSpec((bm, np_), lambda i: (i, 0)),
        out_shape=jax.ShapeDtypeStruct((mp, np_), jnp.float32),
    )(a, b, bias)
    return out[:m, :n]


def _bn(x, g, b):
    m = jnp.mean(x, axis=0)
    v = jnp.var(x, axis=0)
    return (x - m) / jnp.sqrt(v + EPS) * g + b


def _conv(x, eidx, erel, ew, num_rel, p):
    n, d_in = x.shape
    msg = x[eidx[0]] * ew[:, None]
    node_out = eidx[1] * num_rel + erel
    upd = jnp.zeros((n * num_rel, d_in), x.dtype).at[node_out].add(msg)
    upd = upd.reshape(n, num_rel * d_in)
    out = _matmul(upd, p["linW"], p["linb"]) + _matmul(x, p["slW"], p["slb"])
    out = _bn(out, p["bng"], p["bnb"])
    return jax.nn.relu(out)


def kernel(node_feature, edge_index, edge_relation, edge_feature, edge_weight,
           line_edge_index, line_edge_relation, line_edge_weight, node2graph,
           params):
    hiddens = []
    layer_input = node_feature
    edge_input = edge_feature
    for i in range(3):
        hidden = _conv(layer_input, edge_index, edge_relation, edge_weight,
                       NUM_REL, params["node"][i])
        if hidden.shape == layer_input.shape:
            hidden = hidden + layer_input
        edge_hidden = _conv(edge_input, line_edge_index, line_edge_relation,
                            line_edge_weight, NUM_ANGLE, params["edge"][i])
        node_out = edge_index[1] * NUM_REL + edge_relation
        update = jnp.zeros((N * NUM_REL, edge_hidden.shape[1]), jnp.float32
                           ).at[node_out].add(edge_hidden * edge_weight[:, None])
        update = update.reshape(N, NUM_REL * edge_hidden.shape[1])
        update = _matmul(update, params["node"][i]["linW"],
                         params["node"][i]["linb"], relu=True)
        hidden = hidden + update
        edge_input = edge_hidden
        hidden = _bn(hidden, params["bn"][i]["g"], params["bn"][i]["b"])
        hiddens.append(hidden)
        layer_input = hidden
    node_feat = jnp.concatenate(hiddens, axis=-1)
    graph_feat = jax.ops.segment_sum(node_feat, node2graph,
                                     num_segments=NUM_GRAPHS)
    return graph_feat, node_feat
